# baseline (speedup 1.0000x reference)
import functools

import jax
import jax.numpy as jnp
from jax import lax
from jax.experimental import pallas as pl
from jax.experimental.pallas import tpu as pltpu
from jax.experimental.pallas import tpu_sc as plsc

_L = 200
_EMB = 64
_ROWS_PER_WORKER = 8
_N_WORKERS_USED = _L // _ROWS_PER_WORKER
_NUM_CORES = 2

_mesh = plsc.VectorSubcoreMesh(core_axis_name="c", subcore_axis_name="s")


@functools.partial(
    pl.kernel,
    mesh=_mesh,
    out_type=jax.ShapeDtypeStruct((_L, _EMB), jnp.float32),
    scratch_types=[
        pltpu.VMEM((_ROWS_PER_WORKER,), jnp.int32),
        pltpu.VMEM((_ROWS_PER_WORKER, _EMB), jnp.float32),
        pltpu.SemaphoreType.DMA,
    ],
    compiler_params=pltpu.CompilerParams(use_tc_tiling_on_sc=False),
)
def _gather_rows(idx_hbm, table_hbm, out_hbm, idx_v, rows_v, sem):
    wid = lax.axis_index("s") * _NUM_CORES + lax.axis_index("c")

    @pl.when(wid < _N_WORKERS_USED)
    def _():
        base = pl.multiple_of(wid * _ROWS_PER_WORKER, _ROWS_PER_WORKER)
        pltpu.sync_copy(idx_hbm.at[pl.ds(base, _ROWS_PER_WORKER)], idx_v)
        pltpu.async_copy(table_hbm.at[idx_v], rows_v, sem).wait()
        pltpu.sync_copy(rows_v, out_hbm.at[pl.ds(base, _ROWS_PER_WORKER)])


def kernel(words, table, W, b):
    del W, b
    idx = words.astype(jnp.int32)
    rows = _gather_rows(idx, table)
    return rows.reshape(1, _L * _EMB)


# speedup vs baseline: 24.2724x; 24.2724x over previous
import functools

import jax
import jax.numpy as jnp
from jax import lax
from jax.experimental import pallas as pl
from jax.experimental.pallas import tpu as pltpu
from jax.experimental.pallas import tpu_sc as plsc

_L = 200
_EMB = 64
_ROWS_PER_WORKER = 8
_N_WORKERS_USED = _L // _ROWS_PER_WORKER
_NUM_CORES = 2
_LANES = 16

_mesh = plsc.VectorSubcoreMesh(core_axis_name="c", subcore_axis_name="s")


@functools.partial(
    pl.kernel,
    mesh=_mesh,
    out_type=jax.ShapeDtypeStruct((_L, _EMB), jnp.float32),
    scratch_types=[
        pltpu.VMEM((_LANES,), jnp.int32),
        pltpu.VMEM((_ROWS_PER_WORKER, _EMB, 128), jnp.float32),
        pltpu.VMEM((_ROWS_PER_WORKER, _EMB), jnp.float32),
        pltpu.SemaphoreType.DMA,
    ],
    compiler_params=pltpu.CompilerParams(
        use_tc_tiling_on_sc=True, needs_layout_passes=False
    ),
)
def _gather_cols(idx_hbm, tblT_hbm, out_hbm, idx_v, blk_v, rows_v, sem):
    wid = lax.axis_index("s") * _NUM_CORES + lax.axis_index("c")

    @pl.when(wid < _N_WORKERS_USED)
    def _():
        base = pl.multiple_of(wid * _ROWS_PER_WORKER, _ROWS_PER_WORKER)
        pltpu.sync_copy(
            idx_hbm.at[pl.ds(base, _ROWS_PER_WORKER)],
            idx_v.at[pl.ds(0, _ROWS_PER_WORKER)],
        )
        idxvec = idx_v[...]

        copies = []
        for j in range(_ROWS_PER_WORKER):
            w = idxvec[j]
            blk = pl.multiple_of((w // 128) * 128, 128)
            copies.append(
                pltpu.async_copy(
                    tblT_hbm.at[:, pl.ds(blk, 128)], blk_v.at[j], sem
                )
            )
        for c in copies:
            c.wait()

        for j in range(_ROWS_PER_WORKER):
            col = jnp.full((_LANES,), idxvec[j] % 128, jnp.int32)
            jv = jnp.full((_LANES,), j, jnp.int32)
            for k in range(_EMB // _LANES):
                rows = lax.iota(jnp.int32, _LANES) + (k * _LANES)
                vals = plsc.load_gather(blk_v, [jv, rows, col])
                rows_v[j, pl.ds(k * _LANES, _LANES)] = vals

        pltpu.sync_copy(rows_v, out_hbm.at[pl.ds(base, _ROWS_PER_WORKER)])


def kernel(words, table, W, b):
    del W, b
    idx = words.astype(jnp.int32)
    rows = _gather_cols(idx, table.T)
    return rows.reshape(1, _L * _EMB)


# speedup vs baseline: 26.1512x; 1.0774x over previous
import functools

import jax
import jax.numpy as jnp
from jax import lax
from jax.experimental import pallas as pl
from jax.experimental.pallas import tpu as pltpu
from jax.experimental.pallas import tpu_sc as plsc

_L = 200
_EMB = 64
_ROWS_PER_WORKER = 8
_N_WORKERS_USED = _L // _ROWS_PER_WORKER
_NUM_CORES = 2
_LANES = 16

_mesh = plsc.VectorSubcoreMesh(core_axis_name="c", subcore_axis_name="s")


@functools.partial(
    pl.kernel,
    mesh=_mesh,
    out_type=jax.ShapeDtypeStruct((1, _L * _EMB), jnp.float32),
    scratch_types=[
        pltpu.VMEM((_LANES,), jnp.int32),
        pltpu.VMEM((_ROWS_PER_WORKER, _EMB, 128), jnp.float32),
        pltpu.VMEM((_ROWS_PER_WORKER * _EMB,), jnp.float32),
        pltpu.SemaphoreType.DMA,
    ],
    compiler_params=pltpu.CompilerParams(
        use_tc_tiling_on_sc=True, needs_layout_passes=False
    ),
)
def _gather_cols(idx_hbm, tblT_hbm, out_hbm, idx_v, blk_v, rows_v, sem):
    wid = lax.axis_index("s") * _NUM_CORES + lax.axis_index("c")

    @pl.when(wid < _N_WORKERS_USED)
    def _():
        base = pl.multiple_of(wid * _ROWS_PER_WORKER, _ROWS_PER_WORKER)
        pltpu.sync_copy(
            idx_hbm.at[pl.ds(base, _ROWS_PER_WORKER)],
            idx_v.at[pl.ds(0, _ROWS_PER_WORKER)],
        )
        idxvec = idx_v[...]

        copies = []
        for j in range(_ROWS_PER_WORKER):
            w = idxvec[j]
            blk = pl.multiple_of((w // 128) * 128, 128)
            copies.append(
                pltpu.async_copy(
                    tblT_hbm.at[:, pl.ds(blk, 128)], blk_v.at[j], sem
                )
            )
        for c in copies:
            c.wait()

        for j in range(_ROWS_PER_WORKER):
            col = jnp.full((_LANES,), idxvec[j] % 128, jnp.int32)
            jv = jnp.full((_LANES,), j, jnp.int32)
            for k in range(_EMB // _LANES):
                rows = lax.iota(jnp.int32, _LANES) + (k * _LANES)
                vals = plsc.load_gather(blk_v, [jv, rows, col])
                rows_v[pl.ds(j * _EMB + k * _LANES, _LANES)] = vals

        pltpu.sync_copy(
            rows_v,
            out_hbm.at[0, pl.ds(base * _EMB, _ROWS_PER_WORKER * _EMB)],
        )


def kernel(words, table, W, b):
    del W, b
    idx = words.astype(jnp.int32)
    return _gather_cols(idx, table.T)
